# Initial kernel scaffold; baseline (speedup 1.0000x reference)
import jax
import jax.numpy as jnp
from jax.experimental import pallas as pl


def kernel(node_feat, edge_index, edge_feat, W1, b1, W2, b2):
    raise NotImplementedError("write your pallas kernel here")


# speedup vs baseline: 4.9989x; 4.9989x over previous
import functools

import jax
import jax.numpy as jnp
from jax import lax
from jax.experimental import pallas as pl
from jax.experimental.pallas import tpu as pltpu
from jax.experimental.pallas import tpu_sc as plsc

N = 10000
E = 320000
D = 128

NC = 2
NS = 16
NW = NC * NS
EPW = E // NW
CHUNK = 128
NFULL = EPW // CHUNK
TAIL = EPW - NFULL * CHUNK
RPT = 632


def _sc_body(node_hbm, src_hbm, dst_hbm, ef_hbm, out_hbm,
             src_v, dst_v, rows_v, ef_v,
             src_t, dst_t, rows_t, ef_t,
             accum, sem_g, sem_e):
    c = lax.axis_index("c")
    s = lax.axis_index("s")
    w = c * NS + s
    row0 = pl.multiple_of(jnp.minimum(s * RPT, N - RPT), 8)

    pltpu.sync_copy(node_hbm.at[pl.ds(row0, RPT)], accum.at[pl.ds(row0, RPT)])
    plsc.subcore_barrier()

    base = w * EPW

    def chunk_body(i, carry):
        off = pl.multiple_of(base + i * CHUNK, 8)
        pltpu.sync_copy(src_hbm.at[pl.ds(off, CHUNK)], src_v)
        pltpu.sync_copy(dst_hbm.at[pl.ds(off, CHUNK)], dst_v)
        g = pltpu.async_copy(node_hbm.at[src_v], rows_v, sem_g)
        e = pltpu.async_copy(ef_hbm.at[pl.ds(off, CHUNK)], ef_v, sem_e)
        g.wait()
        e.wait()
        pltpu.sync_copy(rows_v, accum.at[dst_v], add=True)
        pltpu.sync_copy(ef_v, accum.at[dst_v], add=True)
        return carry

    lax.fori_loop(0, NFULL, chunk_body, 0)

    toff = pl.multiple_of(base + NFULL * CHUNK, 8)
    pltpu.sync_copy(src_hbm.at[pl.ds(toff, TAIL)], src_t)
    pltpu.sync_copy(dst_hbm.at[pl.ds(toff, TAIL)], dst_t)
    g = pltpu.async_copy(node_hbm.at[src_t], rows_t, sem_g)
    e = pltpu.async_copy(ef_hbm.at[pl.ds(toff, TAIL)], ef_t, sem_e)
    g.wait()
    e.wait()
    pltpu.sync_copy(rows_t, accum.at[dst_t], add=True)
    pltpu.sync_copy(ef_t, accum.at[dst_t], add=True)

    plsc.subcore_barrier()
    pltpu.sync_copy(accum.at[pl.ds(row0, RPT)], out_hbm.at[c, pl.ds(row0, RPT)])


def _sc_partials(node_feat, src, dst, edge_feat):
    mesh = plsc.VectorSubcoreMesh(core_axis_name="c", subcore_axis_name="s",
                                  num_cores=NC, num_subcores=NS)
    return pl.kernel(
        _sc_body,
        out_type=jax.ShapeDtypeStruct((NC, N, D), jnp.float32),
        mesh=mesh,
        scratch_types=[
            pltpu.VMEM((CHUNK,), jnp.int32),
            pltpu.VMEM((CHUNK,), jnp.int32),
            pltpu.VMEM((CHUNK, D), jnp.float32),
            pltpu.VMEM((CHUNK, D), jnp.float32),
            pltpu.VMEM((TAIL,), jnp.int32),
            pltpu.VMEM((TAIL,), jnp.int32),
            pltpu.VMEM((TAIL, D), jnp.float32),
            pltpu.VMEM((TAIL, D), jnp.float32),
            pltpu.VMEM_SHARED((N, D), jnp.float32),
            pltpu.SemaphoreType.DMA,
            pltpu.SemaphoreType.DMA,
        ],
    )(node_feat, src, dst, edge_feat)


BR = 2000


def _mlp_body(p_ref, w1_ref, b1_ref, w2_ref, b2_ref, o_ref):
    h = p_ref[0] + p_ref[1]
    h1 = jnp.dot(h, w1_ref[...], preferred_element_type=jnp.float32) + b1_ref[...]
    h1 = jnp.where(h1 >= 0, h1, 0.01 * h1)
    o_ref[...] = jnp.dot(h1, w2_ref[...], preferred_element_type=jnp.float32) + b2_ref[...]


def _mlp(partials, W1, b1, W2, b2):
    grid = (N // BR,)
    return pl.pallas_call(
        _mlp_body,
        grid=grid,
        in_specs=[
            pl.BlockSpec((NC, BR, D), lambda i: (0, i, 0)),
            pl.BlockSpec((D, D), lambda i: (0, 0)),
            pl.BlockSpec((1, D), lambda i: (0, 0)),
            pl.BlockSpec((D, D), lambda i: (0, 0)),
            pl.BlockSpec((1, D), lambda i: (0, 0)),
        ],
        out_specs=pl.BlockSpec((BR, D), lambda i: (i, 0)),
        out_shape=jax.ShapeDtypeStruct((N, D), jnp.float32),
    )(partials, W1, b1.reshape(1, D), W2, b2.reshape(1, D))


@jax.jit
def kernel(node_feat, edge_index, edge_feat, W1, b1, W2, b2):
    ei = edge_index.astype(jnp.int32)
    partials = _sc_partials(node_feat, ei[0], ei[1], edge_feat)
    return _mlp(partials, W1, b1, W2, b2)


# speedup vs baseline: 8.1835x; 1.6370x over previous
import functools

import jax
import jax.numpy as jnp
from jax import lax
from jax.experimental import pallas as pl
from jax.experimental.pallas import tpu as pltpu
from jax.experimental.pallas import tpu_sc as plsc

N = 10000
E = 320000
D = 128
DG = D // 16

NC = 2
NS = 16
NW = NC * NS
EPW = E // NW
CHUNK = 80
NFULL = EPW // CHUNK

RPT = 632


def _fuse_add(ef, rows, nrows):
    def row_body(r, carry):
        for g in range(DG):
            sl = pl.ds(g * 16, 16)
            plsc.addupdate(ef.at[r, sl], rows[r, sl])
        return carry
    lax.fori_loop(0, nrows, row_body, 0)


def _copy_idx_regs(src_ref, dst_ref):
    for g in range(CHUNK // 16):
        sl = pl.ds(g * 16, 16)
        dst_ref[sl] = src_ref[sl]


def _sc_body(node_hbm, src_hbm, dst_hbm, ef_hbm, out_hbm,
             src0, src1, dst0, dst1, dsc0, dsc1,
             rows0, rows1, ef0, ef1,
             accum,
             sem_i0, sem_i1, sem_g0, sem_g1, sem_e0, sem_e1, sem_s0, sem_s1):
    src_v = (src0, src1)
    dst_v = (dst0, dst1)
    dsc_v = (dsc0, dsc1)
    rows_v = (rows0, rows1)
    ef_v = (ef0, ef1)
    sem_i = (sem_i0, sem_i1)
    sem_g = (sem_g0, sem_g1)
    sem_e = (sem_e0, sem_e1)
    sem_s = (sem_s0, sem_s1)

    c = lax.axis_index("c")
    s = lax.axis_index("s")
    w = c * NS + s
    row0 = pl.multiple_of(jnp.minimum(s * RPT, N - RPT), 8)

    pltpu.sync_copy(node_hbm.at[pl.ds(row0, RPT)], accum.at[pl.ds(row0, RPT)])
    plsc.subcore_barrier()

    base = w * EPW

    def coff(i):
        return pl.multiple_of(base + i * CHUNK, 8)

    def issue_loads(i, b):
        pltpu.async_copy(node_hbm.at[src_v[b]], rows_v[b], sem_g[b])
        pltpu.async_copy(ef_hbm.at[pl.ds(coff(i), CHUNK)], ef_v[b], sem_e[b])

    def drain_loads(i, b):
        pltpu.make_async_copy(node_hbm.at[src_v[b]], rows_v[b], sem_g[b]).wait()
        pltpu.make_async_copy(ef_hbm.at[pl.ds(coff(i), CHUNK)], ef_v[b], sem_e[b]).wait()

    def issue_idx(i, b):
        pltpu.async_copy(src_hbm.at[pl.ds(coff(i), CHUNK)], src_v[b], sem_i[b])
        pltpu.async_copy(dst_hbm.at[pl.ds(coff(i), CHUNK)], dst_v[b], sem_i[b])

    def drain_idx(i, b):
        pltpu.make_async_copy(src_hbm.at[pl.ds(coff(i), CHUNK)], src_v[b], sem_i[b]).wait()
        pltpu.make_async_copy(dst_hbm.at[pl.ds(coff(i), CHUNK)], dst_v[b], sem_i[b]).wait()

    def fuse_and_scatter(b):
        _fuse_add(ef_v[b], rows_v[b], CHUNK)
        _copy_idx_regs(dst_v[b], dsc_v[b])
        pltpu.async_copy(ef_v[b], accum.at[dsc_v[b]], sem_s[b], add=True)

    def drain_scatter(b):
        pltpu.make_async_copy(ef_v[b], accum.at[dsc_v[b]], sem_s[b]).wait()

    pltpu.sync_copy(src_hbm.at[pl.ds(coff(0), CHUNK)], src_v[0])
    pltpu.sync_copy(dst_hbm.at[pl.ds(coff(0), CHUNK)], dst_v[0])
    issue_loads(0, 0)
    issue_idx(1, 1)

    def pair_body(j, carry):
        i0 = 1 + 2 * j

        @pl.when(j > 0)
        def _():
            drain_scatter(1)
        drain_idx(i0, 1)
        issue_loads(i0, 1)
        drain_loads(i0 - 1, 0)
        fuse_and_scatter(0)
        issue_idx(i0 + 1, 0)

        i1 = i0 + 1
        drain_scatter(0)
        drain_idx(i1, 0)
        issue_loads(i1, 0)
        drain_loads(i1 - 1, 1)
        fuse_and_scatter(1)
        issue_idx(i1 + 1, 1)
        return carry

    lax.fori_loop(0, (NFULL - 3) // 2, pair_body, 0)

    drain_scatter(1)
    drain_idx(123, 1)
    issue_loads(123, 1)
    drain_loads(122, 0)
    fuse_and_scatter(0)
    issue_idx(124, 0)

    drain_scatter(0)
    drain_idx(124, 0)
    issue_loads(124, 0)
    drain_loads(123, 1)
    fuse_and_scatter(1)

    drain_loads(124, 0)
    fuse_and_scatter(0)
    drain_scatter(1)
    drain_scatter(0)

    plsc.subcore_barrier()
    pltpu.sync_copy(accum.at[pl.ds(row0, RPT)], out_hbm.at[c, pl.ds(row0, RPT)])


def _sc_partials(node_feat, src, dst, edge_feat):
    mesh = plsc.VectorSubcoreMesh(core_axis_name="c", subcore_axis_name="s",
                                  num_cores=NC, num_subcores=NS)
    return pl.kernel(
        _sc_body,
        out_type=jax.ShapeDtypeStruct((NC, N, D), jnp.float32),
        mesh=mesh,
        scratch_types=[
            pltpu.VMEM((CHUNK,), jnp.int32),
            pltpu.VMEM((CHUNK,), jnp.int32),
            pltpu.VMEM((CHUNK,), jnp.int32),
            pltpu.VMEM((CHUNK,), jnp.int32),
            pltpu.VMEM((CHUNK,), jnp.int32),
            pltpu.VMEM((CHUNK,), jnp.int32),
            pltpu.VMEM((CHUNK, D), jnp.float32),
            pltpu.VMEM((CHUNK, D), jnp.float32),
            pltpu.VMEM((CHUNK, D), jnp.float32),
            pltpu.VMEM((CHUNK, D), jnp.float32),
            pltpu.VMEM_SHARED((N, D), jnp.float32),
            pltpu.SemaphoreType.DMA,
            pltpu.SemaphoreType.DMA,
            pltpu.SemaphoreType.DMA,
            pltpu.SemaphoreType.DMA,
            pltpu.SemaphoreType.DMA,
            pltpu.SemaphoreType.DMA,
            pltpu.SemaphoreType.DMA,
            pltpu.SemaphoreType.DMA,
        ],
    )(node_feat, src, dst, edge_feat)


BR = 2000


def _mlp_body(p_ref, w1_ref, b1_ref, w2_ref, b2_ref, o_ref):
    h = p_ref[0] + p_ref[1]
    h1 = jnp.dot(h, w1_ref[...], preferred_element_type=jnp.float32) + b1_ref[...]
    h1 = jnp.where(h1 >= 0, h1, 0.01 * h1)
    o_ref[...] = jnp.dot(h1, w2_ref[...], preferred_element_type=jnp.float32) + b2_ref[...]


def _mlp(partials, W1, b1, W2, b2):
    grid = (N // BR,)
    return pl.pallas_call(
        _mlp_body,
        grid=grid,
        in_specs=[
            pl.BlockSpec((NC, BR, D), lambda i: (0, i, 0)),
            pl.BlockSpec((D, D), lambda i: (0, 0)),
            pl.BlockSpec((1, D), lambda i: (0, 0)),
            pl.BlockSpec((D, D), lambda i: (0, 0)),
            pl.BlockSpec((1, D), lambda i: (0, 0)),
        ],
        out_specs=pl.BlockSpec((BR, D), lambda i: (i, 0)),
        out_shape=jax.ShapeDtypeStruct((N, D), jnp.float32),
    )(partials, W1, b1.reshape(1, D), W2, b2.reshape(1, D))


@jax.jit
def kernel(node_feat, edge_index, edge_feat, W1, b1, W2, b2):
    ei = edge_index.astype(jnp.int32)
    partials = _sc_partials(node_feat, ei[0], ei[1], edge_feat)
    return _mlp(partials, W1, b1, W2, b2)


# speedup vs baseline: 8.3543x; 1.0209x over previous
import functools

import jax
import jax.numpy as jnp
from jax import lax
from jax.experimental import pallas as pl
from jax.experimental.pallas import tpu as pltpu
from jax.experimental.pallas import tpu_sc as plsc

N = 10000
E = 320000
D = 128
DG = D // 16

NC = 2
NS = 16
NW = NC * NS
EPW = E // NW
CHUNK = 96
NFULL = EPW // CHUNK
TAIL = EPW - NFULL * CHUNK

RPT = 632


def _fuse_add(ef, rows, nrows):
    def row_body(r, carry):
        for g in range(DG):
            sl = pl.ds(g * 16, 16)
            plsc.addupdate(ef.at[r, sl], rows[r, sl])
        return carry
    lax.fori_loop(0, nrows, row_body, 0)


def _copy_idx_regs(src_ref, dst_ref):
    for g in range(CHUNK // 16):
        sl = pl.ds(g * 16, 16)
        dst_ref[sl] = src_ref[sl]


def _sc_body(node_hbm, src_hbm, dst_hbm, ef_hbm, out_hbm,
             src0, src1, dst0, dst1, dsc0, dsc1,
             rows0, rows1, ef0, ef1,
             src_t, dst_t,
             accum,
             sem_i0, sem_i1, sem_g0, sem_g1, sem_e0, sem_e1, sem_s0, sem_s1):
    src_v = (src0, src1)
    dst_v = (dst0, dst1)
    dsc_v = (dsc0, dsc1)
    rows_v = (rows0, rows1)
    ef_v = (ef0, ef1)
    sem_i = (sem_i0, sem_i1)
    sem_g = (sem_g0, sem_g1)
    sem_e = (sem_e0, sem_e1)
    sem_s = (sem_s0, sem_s1)

    c = lax.axis_index("c")
    s = lax.axis_index("s")
    w = c * NS + s
    row0 = pl.multiple_of(jnp.minimum(s * RPT, N - RPT), 8)

    pltpu.sync_copy(node_hbm.at[pl.ds(row0, RPT)], accum.at[pl.ds(row0, RPT)])
    plsc.subcore_barrier()

    base = w * EPW

    def coff(i):
        return pl.multiple_of(base + i * CHUNK, 8)

    def issue_loads(i, b):
        pltpu.async_copy(node_hbm.at[src_v[b]], rows_v[b], sem_g[b])
        pltpu.async_copy(ef_hbm.at[pl.ds(coff(i), CHUNK)], ef_v[b], sem_e[b])

    def drain_loads(i, b):
        pltpu.make_async_copy(node_hbm.at[src_v[b]], rows_v[b], sem_g[b]).wait()
        pltpu.make_async_copy(ef_hbm.at[pl.ds(coff(i), CHUNK)], ef_v[b], sem_e[b]).wait()

    def issue_idx(i, b):
        pltpu.async_copy(src_hbm.at[pl.ds(coff(i), CHUNK)], src_v[b], sem_i[b])
        pltpu.async_copy(dst_hbm.at[pl.ds(coff(i), CHUNK)], dst_v[b], sem_i[b])

    def drain_idx(i, b):
        pltpu.make_async_copy(src_hbm.at[pl.ds(coff(i), CHUNK)], src_v[b], sem_i[b]).wait()
        pltpu.make_async_copy(dst_hbm.at[pl.ds(coff(i), CHUNK)], dst_v[b], sem_i[b]).wait()

    def fuse_and_scatter(b):
        _fuse_add(ef_v[b], rows_v[b], CHUNK)
        _copy_idx_regs(dst_v[b], dsc_v[b])
        pltpu.async_copy(ef_v[b], accum.at[dsc_v[b]], sem_s[b], add=True)

    def drain_scatter(b):
        pltpu.make_async_copy(ef_v[b], accum.at[dsc_v[b]], sem_s[b]).wait()

    pltpu.sync_copy(src_hbm.at[pl.ds(coff(0), CHUNK)], src_v[0])
    pltpu.sync_copy(dst_hbm.at[pl.ds(coff(0), CHUNK)], dst_v[0])
    issue_loads(0, 0)
    issue_idx(1, 1)

    def pair_body(j, carry):
        i0 = 1 + 2 * j

        @pl.when(j > 0)
        def _():
            drain_scatter(1)
        drain_idx(i0, 1)
        issue_loads(i0, 1)
        drain_loads(i0 - 1, 0)
        fuse_and_scatter(0)
        issue_idx(i0 + 1, 0)

        i1 = i0 + 1
        drain_scatter(0)
        drain_idx(i1, 0)
        issue_loads(i1, 0)
        drain_loads(i1 - 1, 1)
        fuse_and_scatter(1)
        issue_idx(i1 + 1, 1)
        return carry

    lax.fori_loop(0, (NFULL - 2) // 2, pair_body, 0)

    drain_scatter(1)
    drain_idx(NFULL - 1, 1)
    issue_loads(NFULL - 1, 1)
    drain_loads(NFULL - 2, 0)
    fuse_and_scatter(0)

    drain_loads(NFULL - 1, 1)
    fuse_and_scatter(1)

    drain_scatter(0)
    toff = pl.multiple_of(base + NFULL * CHUNK, 8)
    pltpu.sync_copy(src_hbm.at[pl.ds(toff, TAIL)], src_t)
    pltpu.sync_copy(dst_hbm.at[pl.ds(toff, TAIL)], dst_t)
    pltpu.async_copy(node_hbm.at[src_t], rows_v[0].at[pl.ds(0, TAIL)], sem_g[0])
    pltpu.async_copy(ef_hbm.at[pl.ds(toff, TAIL)], ef_v[0].at[pl.ds(0, TAIL)], sem_e[0])
    pltpu.make_async_copy(node_hbm.at[src_t], rows_v[0].at[pl.ds(0, TAIL)], sem_g[0]).wait()
    pltpu.make_async_copy(ef_hbm.at[pl.ds(toff, TAIL)], ef_v[0].at[pl.ds(0, TAIL)], sem_e[0]).wait()
    _fuse_add(ef_v[0], rows_v[0], TAIL)
    pltpu.sync_copy(ef_v[0].at[pl.ds(0, TAIL)], accum.at[dst_t], add=True)

    drain_scatter(1)

    plsc.subcore_barrier()
    pltpu.sync_copy(accum.at[pl.ds(row0, RPT)], out_hbm.at[c, pl.ds(row0, RPT)])


def _sc_partials(node_feat, src, dst, edge_feat):
    mesh = plsc.VectorSubcoreMesh(core_axis_name="c", subcore_axis_name="s",
                                  num_cores=NC, num_subcores=NS)
    return pl.kernel(
        _sc_body,
        out_type=jax.ShapeDtypeStruct((NC, N, D), jnp.float32),
        mesh=mesh,
        scratch_types=[
            pltpu.VMEM((CHUNK,), jnp.int32),
            pltpu.VMEM((CHUNK,), jnp.int32),
            pltpu.VMEM((CHUNK,), jnp.int32),
            pltpu.VMEM((CHUNK,), jnp.int32),
            pltpu.VMEM((CHUNK,), jnp.int32),
            pltpu.VMEM((CHUNK,), jnp.int32),
            pltpu.VMEM((CHUNK, D), jnp.float32),
            pltpu.VMEM((CHUNK, D), jnp.float32),
            pltpu.VMEM((CHUNK, D), jnp.float32),
            pltpu.VMEM((CHUNK, D), jnp.float32),
            pltpu.VMEM((TAIL,), jnp.int32),
            pltpu.VMEM((TAIL,), jnp.int32),
            pltpu.VMEM_SHARED((N, D), jnp.float32),
            pltpu.SemaphoreType.DMA,
            pltpu.SemaphoreType.DMA,
            pltpu.SemaphoreType.DMA,
            pltpu.SemaphoreType.DMA,
            pltpu.SemaphoreType.DMA,
            pltpu.SemaphoreType.DMA,
            pltpu.SemaphoreType.DMA,
            pltpu.SemaphoreType.DMA,
        ],
    )(node_feat, src, dst, edge_feat)


BR = 2000


def _mlp_body(p_ref, w1_ref, b1_ref, w2_ref, b2_ref, o_ref):
    h = p_ref[0] + p_ref[1]
    h1 = jnp.dot(h, w1_ref[...], preferred_element_type=jnp.float32) + b1_ref[...]
    h1 = jnp.where(h1 >= 0, h1, 0.01 * h1)
    o_ref[...] = jnp.dot(h1, w2_ref[...], preferred_element_type=jnp.float32) + b2_ref[...]


def _mlp(partials, W1, b1, W2, b2):
    grid = (N // BR,)
    return pl.pallas_call(
        _mlp_body,
        grid=grid,
        in_specs=[
            pl.BlockSpec((NC, BR, D), lambda i: (0, i, 0)),
            pl.BlockSpec((D, D), lambda i: (0, 0)),
            pl.BlockSpec((1, D), lambda i: (0, 0)),
            pl.BlockSpec((D, D), lambda i: (0, 0)),
            pl.BlockSpec((1, D), lambda i: (0, 0)),
        ],
        out_specs=pl.BlockSpec((BR, D), lambda i: (i, 0)),
        out_shape=jax.ShapeDtypeStruct((N, D), jnp.float32),
    )(partials, W1, b1.reshape(1, D), W2, b2.reshape(1, D))


@jax.jit
def kernel(node_feat, edge_index, edge_feat, W1, b1, W2, b2):
    ei = edge_index.astype(jnp.int32)
    partials = _sc_partials(node_feat, ei[0], ei[1], edge_feat)
    return _mlp(partials, W1, b1, W2, b2)
